# baseline (speedup 1.0000x reference)
import functools
import jax
import jax.numpy as jnp
from jax import lax
from jax.experimental import pallas as pl
from jax.experimental.pallas import tpu as pltpu
from jax.experimental.pallas import tpu_sc as plsc

_N = 10000
_E = 320000
_FIN = 128
_H = 64
_C = 10
_G = 64

_NC = 2
_NS = 16
_NW = _NC * _NS
_B = 128
_EPT = 10240
_NCH = _EPT // _B
_EPAD = _EPT * _NW
_PAD = 8
_NPAD = _N + _PAD
_RPT = _N // _NS
_DEGW = 16

_mesh = plsc.VectorSubcoreMesh(core_axis_name="c", subcore_axis_name="s")
_sc_params = pltpu.CompilerParams(use_tc_tiling_on_sc=False)


@functools.partial(
    pl.kernel,
    out_type=jax.ShapeDtypeStruct((_NC, _NS, _RPT, _DEGW), jnp.float32),
    mesh=_mesh,
    scratch_types=[
        pltpu.VMEM((_NCH, _B), jnp.int32),
        pltpu.VMEM((_B, _DEGW), jnp.float32),
        pltpu.VMEM_SHARED((_NPAD, _DEGW), jnp.float32),
        pltpu.SemaphoreType.DMA,
    ],
    compiler_params=_sc_params,
)
def _deg_kernel(dst_hbm, zeros_hbm, ones_hbm, out_hbm, dst_v, ones_v, acc_sh,
                sem):
    c = lax.axis_index("c")
    s = lax.axis_index("s")
    pltpu.sync_copy(zeros_hbm, acc_sh.at[pl.ds(s * _RPT, _RPT)])
    pltpu.sync_copy(ones_hbm, ones_v)
    pltpu.sync_copy(dst_hbm.at[c, s], dst_v)
    plsc.subcore_barrier()

    def body(j, carry):
        pltpu.async_copy(ones_v, acc_sh.at[dst_v.at[j]], sem, add=True)
        return carry

    lax.fori_loop(0, _NCH, body, 0)

    def drain(j, carry):
        pltpu.make_async_copy(ones_v, acc_sh.at[dst_v.at[0]], sem).wait()
        return carry

    lax.fori_loop(0, _NCH, drain, 0)
    plsc.subcore_barrier()
    pltpu.sync_copy(acc_sh.at[pl.ds(s * _RPT, _RPT)], out_hbm.at[c, s])


@functools.partial(
    pl.kernel,
    out_type=jax.ShapeDtypeStruct((_NC, _NS, _RPT, _H), jnp.float32),
    mesh=_mesh,
    scratch_types=[
        pltpu.VMEM((_NCH, _B), jnp.int32),
        pltpu.VMEM((_NCH, _B), jnp.int32),
        pltpu.VMEM((_B, _H), jnp.float32),
        pltpu.VMEM((_B, _H), jnp.float32),
        pltpu.VMEM_SHARED((_NPAD, _H), jnp.float32),
        pltpu.SemaphoreType.DMA,
        pltpu.SemaphoreType.DMA,
    ],
    compiler_params=_sc_params,
)
def _scatter_kernel(g_hbm, src_hbm, dst_hbm, zeros_hbm, out_hbm,
                    src_v, dst_v, rows0, rows1, acc_sh, sem0, sem1):
    c = lax.axis_index("c")
    s = lax.axis_index("s")
    pltpu.sync_copy(zeros_hbm, acc_sh.at[pl.ds(s * _RPT, _RPT)])
    pltpu.sync_copy(src_hbm.at[c, s], src_v)
    pltpu.sync_copy(dst_hbm.at[c, s], dst_v)
    plsc.subcore_barrier()

    pltpu.async_copy(g_hbm.at[src_v.at[0]], rows0, sem0)
    pltpu.async_copy(g_hbm.at[src_v.at[1]], rows1, sem1)

    def step(j, buf, sem, prefetch):
        pltpu.make_async_copy(g_hbm.at[src_v.at[j]], buf, sem).wait()
        pltpu.sync_copy(buf, acc_sh.at[dst_v.at[j]], add=True)
        if prefetch:
            pltpu.async_copy(g_hbm.at[src_v.at[j + 2]], buf, sem)

    def body(i, carry):
        j = 2 * i
        step(j, rows0, sem0, True)
        step(j + 1, rows1, sem1, True)
        return carry

    lax.fori_loop(0, _NCH // 2 - 1, body, 0)
    step(_NCH - 2, rows0, sem0, False)
    step(_NCH - 1, rows1, sem1, False)
    plsc.subcore_barrier()
    pltpu.sync_copy(acc_sh.at[pl.ds(s * _RPT, _RPT)], out_hbm.at[c, s])


def _prep_body(x_ref, w1_ref, degp_ref, g_ref, dinv_ref):
    deg = degp_ref[0, :, 0:1] + degp_ref[1, :, 0:1] + 1.0
    dinv = lax.rsqrt(deg)
    g = jnp.dot(x_ref[...], w1_ref[...],
                preferred_element_type=jnp.float32) * dinv
    g_ref[pl.ds(0, _N), :] = g
    g_ref[pl.ds(_N, _PAD), :] = jnp.zeros((_PAD, _H), jnp.float32)
    dinv_ref[...] = dinv


def _mid_body(accp_ref, g_ref, dinv_ref, b_ref, w_ref, gout_ref):
    dinv = dinv_ref[...]
    acc = accp_ref[0] + accp_ref[1] + g_ref[pl.ds(0, _N), :]
    h = jnp.maximum(acc * dinv + b_ref[...], 0.0)
    gout_ref[pl.ds(0, _N), :] = jnp.dot(
        h, w_ref[...], preferred_element_type=jnp.float32) * dinv
    gout_ref[pl.ds(_N, _PAD), :] = jnp.zeros((_PAD, _H), jnp.float32)


def _final_body(accp_ref, g_ref, dinv_ref, b_ref, batch_ref,
                wc1_ref, bc1_ref, wc2_ref, bc2_ref, out_ref):
    dinv = dinv_ref[...]
    acc = accp_ref[0] + accp_ref[1] + g_ref[pl.ds(0, _N), :]
    h = jnp.maximum(acc * dinv + b_ref[...], 0.0)
    seg = batch_ref[...]
    gids = lax.broadcasted_iota(jnp.int32, (_G, _N), 0)
    mask = (jnp.broadcast_to(seg, (_G, _N)) == gids).astype(jnp.float32)
    sums = jnp.dot(mask, h, preferred_element_type=jnp.float32)
    cnt = jnp.sum(mask, axis=1, keepdims=True)
    pooled = sums / jnp.maximum(cnt, 1.0)
    z = jnp.maximum(jnp.dot(pooled, wc1_ref[...],
                            preferred_element_type=jnp.float32) + bc1_ref[...],
                    0.0)
    out_ref[...] = jnp.dot(z, wc2_ref[...],
                           preferred_element_type=jnp.float32) + bc2_ref[...]


_prep_call = pl.pallas_call(
    _prep_body,
    out_shape=(jax.ShapeDtypeStruct((_NPAD, _H), jnp.float32),
               jax.ShapeDtypeStruct((_N, 1), jnp.float32)),
)

_mid_call = pl.pallas_call(
    _mid_body,
    out_shape=jax.ShapeDtypeStruct((_NPAD, _H), jnp.float32),
)

_final_call = pl.pallas_call(
    _final_body,
    out_shape=jax.ShapeDtypeStruct((_G, _C), jnp.float32),
)


@jax.jit
def kernel(x, edge_index, batch, W1, b1, W2, b2, W3, b3, Wc1, bc1, Wc2, bc2):
    pad = jnp.full((_EPAD - _E,), _N, jnp.int32)
    src = jnp.concatenate([edge_index[0], pad]).reshape(_NC, _NS, _NCH, _B)
    dst = jnp.concatenate([edge_index[1], pad]).reshape(_NC, _NS, _NCH, _B)

    zeros_deg = jnp.zeros((_RPT, _DEGW), jnp.float32)
    ones_deg = jnp.ones((_B, _DEGW), jnp.float32)
    zeros_h = jnp.zeros((_RPT, _H), jnp.float32)

    degp = _deg_kernel(dst, zeros_deg, ones_deg).reshape(_NC, _N, _DEGW)
    g1, dinv = _prep_call(x, W1, degp)

    def scatter(g):
        return _scatter_kernel(g, src, dst, zeros_h).reshape(_NC, _N, _H)

    acc1 = scatter(g1)
    g2 = _mid_call(acc1, g1, dinv, b1.reshape(1, _H), W2)
    acc2 = scatter(g2)
    g3 = _mid_call(acc2, g2, dinv, b2.reshape(1, _H), W3)
    acc3 = scatter(g3)

    out = _final_call(acc3, g3, dinv, b3.reshape(1, _H),
                      batch.reshape(1, _N), Wc1, bc1.reshape(1, _H // 2),
                      Wc2, bc2.reshape(1, _C))
    return out


# speedup vs baseline: 2.3484x; 2.3484x over previous
import functools
import jax
import jax.numpy as jnp
from jax import lax
from jax.experimental import pallas as pl
from jax.experimental.pallas import tpu as pltpu
from jax.experimental.pallas import tpu_sc as plsc

_N = 10000
_E = 320000
_FIN = 128
_H = 64
_C = 10
_G = 64

_NC = 2
_NS = 16
_NW = _NC * _NS
_B = 128
_EPT = 10240
_NCH = _EPT // _B
_EPTR = _E // _NW
_PAD = _EPT - _EPTR
_NPAD = _N + _PAD
_RPT = _N // _NS
_GRT = _NPAD // _NS
_DEGW = 16

_mesh = plsc.VectorSubcoreMesh(core_axis_name="c", subcore_axis_name="s")
_sc_params = pltpu.CompilerParams(use_tc_tiling_on_sc=False)


@functools.partial(
    pl.kernel,
    out_type=jax.ShapeDtypeStruct((_NC, _NS, _RPT, _DEGW), jnp.float32),
    mesh=_mesh,
    scratch_types=[
        pltpu.VMEM((_NCH, _B), jnp.int32),
        pltpu.VMEM((_B, _DEGW), jnp.float32),
        pltpu.VMEM_SHARED((_NPAD, _DEGW), jnp.float32),
        pltpu.SemaphoreType.DMA,
        pltpu.SemaphoreType.DMA,
        pltpu.SemaphoreType.DMA,
    ],
    compiler_params=_sc_params,
)
def _deg_kernel(dst_hbm, zeros_hbm, ones_hbm, out_hbm, dst_v, ones_v, acc_sh,
                sem, sem1, sem2):
    c = lax.axis_index("c")
    s = lax.axis_index("s")
    st0 = pltpu.async_copy(zeros_hbm, acc_sh.at[pl.ds(s * _RPT, _RPT)], sem)
    st1 = pltpu.async_copy(ones_hbm, ones_v, sem1)
    st2 = pltpu.async_copy(dst_hbm.at[c, s], dst_v, sem2)
    st0.wait()
    st1.wait()
    st2.wait()
    plsc.subcore_barrier()

    def body(j, carry):
        pltpu.async_copy(ones_v, acc_sh.at[dst_v.at[j]], sem, add=True)
        return carry

    lax.fori_loop(0, _NCH, body, 0)

    def drain(j, carry):
        pltpu.make_async_copy(ones_v, acc_sh.at[dst_v.at[0]], sem).wait()
        return carry

    lax.fori_loop(0, _NCH, drain, 0)
    plsc.subcore_barrier()
    pltpu.sync_copy(acc_sh.at[pl.ds(s * _RPT, _RPT)], out_hbm.at[c, s])


@functools.partial(
    pl.kernel,
    out_type=jax.ShapeDtypeStruct((_NC, _NS, _RPT, _H), jnp.float32),
    mesh=_mesh,
    scratch_types=[
        pltpu.VMEM((_NCH, _B), jnp.int32),
        pltpu.VMEM((_NCH, _B), jnp.int32),
        pltpu.VMEM((_B, _H), jnp.float32),
        pltpu.VMEM((_B, _H), jnp.float32),
        pltpu.VMEM((_B, _H), jnp.float32),
        pltpu.VMEM_SHARED((_NPAD, _H), jnp.float32),
        pltpu.VMEM_SHARED((_NPAD, _H), jnp.float32),
        pltpu.SemaphoreType.DMA,
        pltpu.SemaphoreType.DMA,
        pltpu.SemaphoreType.DMA,
        pltpu.SemaphoreType.DMA,
        pltpu.SemaphoreType.DMA,
        pltpu.SemaphoreType.DMA,
    ],
    compiler_params=_sc_params,
)
def _scatter_kernel(g_hbm, src_hbm, dst_hbm, zeros_hbm, out_hbm,
                    src_v, dst_v, rows0, rows1, rows2, acc_sh, g_sh,
                    sem0, sem1, sem2, sem3, sem4, sem5):
    c = lax.axis_index("c")
    s = lax.axis_index("s")
    st0 = pltpu.async_copy(g_hbm.at[pl.ds(s * _GRT, _GRT)],
                           g_sh.at[pl.ds(s * _GRT, _GRT)], sem0)
    st1 = pltpu.async_copy(zeros_hbm, acc_sh.at[pl.ds(s * _RPT, _RPT)], sem1)
    st2 = pltpu.async_copy(src_hbm.at[c, s], src_v, sem2)
    st3 = pltpu.async_copy(dst_hbm.at[c, s], dst_v, sem3)
    st0.wait()
    st1.wait()
    st2.wait()
    st3.wait()
    plsc.subcore_barrier()

    bufs = (rows0, rows1, rows2)
    sgs = (sem0, sem1, sem2)
    sss = (sem3, sem4, sem5)

    def fire_gather(t, b):
        pltpu.async_copy(g_sh.at[src_v.at[t]], bufs[b], sgs[b])

    def wait_gather(t, b):
        pltpu.make_async_copy(g_sh.at[src_v.at[t]], bufs[b], sgs[b]).wait()

    def fire_scatter(j, b):
        pltpu.async_copy(bufs[b], acc_sh.at[dst_v.at[j]], sss[b], add=True)

    def wait_scatter(b):
        pltpu.make_async_copy(bufs[b], acc_sh.at[dst_v.at[0]], sss[b]).wait()

    fire_gather(0, 0)
    fire_gather(1, 1)
    wait_gather(0, 0)
    fire_scatter(0, 0)
    fire_gather(2, 2)
    wait_gather(1, 1)
    fire_scatter(1, 1)
    wait_scatter(0)
    fire_gather(3, 0)

    def body(i, carry):
        j0 = 3 * i + 2
        for b_off in range(3):
            j = j0 + b_off
            b = (2 + b_off) % 3
            nb = (1 + b_off) % 3
            wait_gather(j, b)
            fire_scatter(j, b)
            wait_scatter(nb)
            fire_gather(j + 2, nb)
        return carry

    lax.fori_loop(0, (_NCH - 5) // 3, body, 0)
    wait_gather(_NCH - 3, (_NCH - 3) % 3)
    fire_scatter(_NCH - 3, (_NCH - 3) % 3)
    wait_scatter((_NCH - 1) % 3)
    fire_gather(_NCH - 1, (_NCH - 1) % 3)
    wait_gather(_NCH - 2, (_NCH - 2) % 3)
    fire_scatter(_NCH - 2, (_NCH - 2) % 3)
    wait_gather(_NCH - 1, (_NCH - 1) % 3)
    fire_scatter(_NCH - 1, (_NCH - 1) % 3)
    wait_scatter((_NCH - 3) % 3)
    wait_scatter((_NCH - 2) % 3)
    wait_scatter((_NCH - 1) % 3)
    plsc.subcore_barrier()
    pltpu.sync_copy(acc_sh.at[pl.ds(s * _RPT, _RPT)], out_hbm.at[c, s])


def _prep_body(x_ref, w1_ref, degp_ref, g_ref, dinv_ref):
    deg = degp_ref[0, :, 0:1] + degp_ref[1, :, 0:1] + 1.0
    dinv = lax.rsqrt(deg)
    g = jnp.dot(x_ref[...], w1_ref[...],
                preferred_element_type=jnp.float32) * dinv
    g_ref[pl.ds(0, _N), :] = g
    g_ref[pl.ds(_N, _PAD), :] = jnp.zeros((_PAD, _H), jnp.float32)
    dinv_ref[...] = dinv


def _mid_body(accp_ref, g_ref, dinv_ref, b_ref, w_ref, gout_ref):
    dinv = dinv_ref[...]
    acc = accp_ref[0] + accp_ref[1] + g_ref[pl.ds(0, _N), :]
    h = jnp.maximum(acc * dinv + b_ref[...], 0.0)
    gout_ref[pl.ds(0, _N), :] = jnp.dot(
        h, w_ref[...], preferred_element_type=jnp.float32) * dinv
    gout_ref[pl.ds(_N, _PAD), :] = jnp.zeros((_PAD, _H), jnp.float32)


def _final_body(accp_ref, g_ref, dinv_ref, b_ref, batch_ref,
                wc1_ref, bc1_ref, wc2_ref, bc2_ref, out_ref):
    dinv = dinv_ref[...]
    acc = accp_ref[0] + accp_ref[1] + g_ref[pl.ds(0, _N), :]
    h = jnp.maximum(acc * dinv + b_ref[...], 0.0)
    seg = batch_ref[...]
    gids = lax.broadcasted_iota(jnp.int32, (_G, _N), 0)
    mask = (jnp.broadcast_to(seg, (_G, _N)) == gids).astype(jnp.float32)
    sums = jnp.dot(mask, h, preferred_element_type=jnp.float32)
    cnt = jnp.sum(mask, axis=1, keepdims=True)
    pooled = sums / jnp.maximum(cnt, 1.0)
    z = jnp.maximum(jnp.dot(pooled, wc1_ref[...],
                            preferred_element_type=jnp.float32) + bc1_ref[...],
                    0.0)
    out_ref[...] = jnp.dot(z, wc2_ref[...],
                           preferred_element_type=jnp.float32) + bc2_ref[...]


_prep_call = pl.pallas_call(
    _prep_body,
    out_shape=(jax.ShapeDtypeStruct((_NPAD, _H), jnp.float32),
               jax.ShapeDtypeStruct((_N, 1), jnp.float32)),
)

_mid_call = pl.pallas_call(
    _mid_body,
    out_shape=jax.ShapeDtypeStruct((_NPAD, _H), jnp.float32),
)

_final_call = pl.pallas_call(
    _final_body,
    out_shape=jax.ShapeDtypeStruct((_G, _C), jnp.float32),
)


@jax.jit
def kernel(x, edge_index, batch, W1, b1, W2, b2, W3, b3, Wc1, bc1, Wc2, bc2):
    pad_s = jnp.full((_NW, _PAD), _N, jnp.int32)
    pad_d = jnp.broadcast_to(_N + jnp.arange(_PAD, dtype=jnp.int32),
                             (_NW, _PAD))
    src = jnp.concatenate([edge_index[0].reshape(_NW, _EPTR), pad_s],
                          axis=1).reshape(_NC, _NS, _NCH, _B)
    dst = jnp.concatenate([edge_index[1].reshape(_NW, _EPTR), pad_d],
                          axis=1).reshape(_NC, _NS, _NCH, _B)

    zeros_deg = jnp.zeros((_RPT, _DEGW), jnp.float32)
    ones_deg = jnp.ones((_B, _DEGW), jnp.float32)
    zeros_h = jnp.zeros((_RPT, _H), jnp.float32)

    degp = _deg_kernel(dst, zeros_deg, ones_deg).reshape(_NC, _N, _DEGW)
    g1, dinv = _prep_call(x, W1, degp)

    def scatter(g):
        return _scatter_kernel(g, src, dst, zeros_h).reshape(_NC, _N, _H)

    acc1 = scatter(g1)
    g2 = _mid_call(acc1, g1, dinv, b1.reshape(1, _H), W2)
    acc2 = scatter(g2)
    g3 = _mid_call(acc2, g2, dinv, b2.reshape(1, _H), W3)
    acc3 = scatter(g3)

    out = _final_call(acc3, g3, dinv, b3.reshape(1, _H),
                      batch.reshape(1, _N), Wc1, bc1.reshape(1, _H // 2),
                      Wc2, bc2.reshape(1, _C))
    return out
